# baseline (speedup 1.0000x reference)
import functools

import jax
import jax.numpy as jnp
from jax import lax
from jax.experimental import pallas as pl
from jax.experimental.pallas import tpu as pltpu
from jax.experimental.pallas import tpu_sc as plsc

N = 10000
D = 128
N_EDGES = 320000

NUM_TILES = 32
CHUNK = 128
PHASES = 5
CHUNKS_PER_PHASE = 16
CHUNKS_PER_TILE = PHASES * CHUNKS_PER_PHASE
E_PER_TILE = CHUNK * CHUNKS_PER_TILE
EPAD = E_PER_TILE * NUM_TILES
NPAD = 10240
ROWS_PER_TILE = NPAD // 16
NBUF = 2

_MESH = plsc.VectorSubcoreMesh(core_axis_name="c", subcore_axis_name="s")


def _sc_agg_body(table, srcp, dstp, z2, out_agg, src_all, dst_all, *rest):
    rows = rest[:NBUF]
    acc_sh, gsem, ssem = rest[NBUF:]

    c = lax.axis_index("c")
    s = lax.axis_index("s")
    wid = c * 16 + s
    row0 = s * ROWS_PER_TILE

    pltpu.sync_copy(z2.at[pl.ds(row0, ROWS_PER_TILE)],
                    acc_sh.at[pl.ds(row0, ROWS_PER_TILE)])
    plsc.subcore_barrier()

    def drain_scatters():
        for b in range(NBUF):
            pltpu.make_async_copy(z2.at[pl.ds(0, CHUNK)], rows[b],
                                  ssem).wait()

    def group(g, carry):
        j0 = g * NBUF

        @pl.when(g > 0)
        def _():
            drain_scatters()

        handles = [
            pltpu.async_copy(table.at[src_all.at[j0 + b]], rows[b], gsem)
            for b in range(NBUF)
        ]
        for b in range(NBUF):
            handles[b].wait()
            pltpu.async_copy(rows[b], acc_sh.at[dst_all.at[j0 + b]], ssem,
                             add=True)
        return carry

    for phase in range(PHASES):
        c0 = wid * CHUNKS_PER_TILE + phase * CHUNKS_PER_PHASE
        pltpu.sync_copy(srcp.at[pl.ds(c0, CHUNKS_PER_PHASE)], src_all)
        pltpu.sync_copy(dstp.at[pl.ds(c0, CHUNKS_PER_PHASE)], dst_all)
        lax.fori_loop(0, CHUNKS_PER_PHASE // NBUF, group, 0)
        drain_scatters()
    plsc.subcore_barrier()

    pltpu.sync_copy(acc_sh.at[pl.ds(row0, ROWS_PER_TILE)],
                    out_agg.at[c, pl.ds(row0, ROWS_PER_TILE)])


_sc_agg = pl.kernel(
    _sc_agg_body,
    mesh=_MESH,
    out_type=jax.ShapeDtypeStruct((2, NPAD, D), jnp.float32),
    scratch_types=(
        [pltpu.VMEM((CHUNKS_PER_PHASE, CHUNK), jnp.int32),
         pltpu.VMEM((CHUNKS_PER_PHASE, CHUNK), jnp.int32)]
        + [pltpu.VMEM((CHUNK, D), jnp.float32) for _ in range(NBUF)]
        + [pltpu.VMEM_SHARED((NPAD, D), jnp.float32),
           pltpu.SemaphoreType.DMA,
           pltpu.SemaphoreType.DMA]
    ),
)


DCHUNK = 128
DCHUNKS = E_PER_TILE // DCHUNK
DNBUF = 8


def _sc_deg_body(dstp2, z1, out_deg, dst_d, ones_v, deg_sh, sem):
    c = lax.axis_index("c")
    s = lax.axis_index("s")
    wid = c * 16 + s
    row0 = s * ROWS_PER_TILE

    for i in range(DCHUNK // 16):
        ones_v[pl.ds(i * 16, 16)] = jnp.ones((16,), jnp.float32)
    pltpu.sync_copy(z1.at[pl.ds(row0, ROWS_PER_TILE)],
                    deg_sh.at[pl.ds(row0, ROWS_PER_TILE)])
    pltpu.sync_copy(dstp2.at[pl.ds(wid * DCHUNKS, DCHUNKS)], dst_d)
    plsc.subcore_barrier()

    def group(g, carry):
        j0 = g * DNBUF
        handles = [
            pltpu.async_copy(ones_v, deg_sh.at[dst_d.at[j0 + b]], sem,
                             add=True)
            for b in range(DNBUF)
        ]
        for h in handles:
            h.wait()
        return carry

    lax.fori_loop(0, DCHUNKS // DNBUF, group, 0)
    plsc.subcore_barrier()
    pltpu.sync_copy(deg_sh.at[pl.ds(row0, ROWS_PER_TILE)],
                    out_deg.at[c, pl.ds(row0, ROWS_PER_TILE)])


_sc_deg = pl.kernel(
    _sc_deg_body,
    mesh=_MESH,
    out_type=jax.ShapeDtypeStruct((2, NPAD), jnp.float32),
    scratch_types=[
        pltpu.VMEM((DCHUNKS, DCHUNK), jnp.int32),
        pltpu.VMEM((DCHUNK,), jnp.float32),
        pltpu.VMEM_SHARED((NPAD,), jnp.float32),
        pltpu.SemaphoreType.DMA,
    ],
)


def _dense_body(h_ref, agg_ref, degT_ref, wt_ref, b_ref, out_ref):
    h = h_ref[...]
    agg = agg_ref[0] + agg_ref[1]
    d = degT_ref[:, 0:1] + degT_ref[:, 1:2]
    mean = agg * (1.0 / jnp.maximum(d, 1.0))
    z = (jnp.dot(h, wt_ref[0:D], preferred_element_type=jnp.float32)
         + jnp.dot(mean, wt_ref[D:2 * D], preferred_element_type=jnp.float32)
         + b_ref[...])
    a = jnp.where(z >= 0, z, 0.01 * z)
    nrm = jnp.sqrt(jnp.sum(a * a, axis=1, keepdims=True))
    out_ref[...] = a / jnp.maximum(nrm, 1e-6)


def _dense(h, agg, degT, Wt, b2d):
    R = 1024
    return pl.pallas_call(
        _dense_body,
        grid=(NPAD // R,),
        in_specs=[
            pl.BlockSpec((R, D), lambda i: (i, 0)),
            pl.BlockSpec((2, R, D), lambda i: (0, i, 0)),
            pl.BlockSpec((R, 2), lambda i: (i, 0)),
            pl.BlockSpec((2 * D, D), lambda i: (0, 0)),
            pl.BlockSpec((1, D), lambda i: (0, 0)),
        ],
        out_specs=pl.BlockSpec((R, D), lambda i: (i, 0)),
        out_shape=jax.ShapeDtypeStruct((N, D), jnp.float32),
    )(h, agg, degT, Wt, b2d)


def kernel(node_ids, edge_index, node_emb, W1, b1, W2, b2):
    h0 = lax.slice(node_emb, (1, 0), (N + 1, D))

    src = edge_index[0]
    dst = edge_index[1]
    npad_e = EPAD - N_EDGES
    pad_iota = jnp.arange(npad_e, dtype=jnp.int32)
    srcp = jnp.concatenate([src, pad_iota % N])
    dstp = jnp.concatenate([dst, N + pad_iota % (NPAD - N)])
    srcp2 = srcp.reshape(EPAD // CHUNK, CHUNK)
    srcp2e = srcp2 + 1
    dstp2 = dstp.reshape(EPAD // CHUNK, CHUNK)
    dstp2d = dstp.reshape(EPAD // DCHUNK, DCHUNK)

    z2 = jnp.zeros((NPAD, D), jnp.float32)
    z1 = jnp.zeros((NPAD,), jnp.float32)

    W1t = W1.T
    W2t = W2.T
    b1r = b1.reshape(1, D)
    b2r = b2.reshape(1, D)

    deg = _sc_deg(dstp2d, z1)
    degT = deg.T
    agg1 = _sc_agg(node_emb, srcp2e, dstp2, z2)
    h1 = _dense(h0, agg1, degT, W1t, b1r)

    agg2 = _sc_agg(h1, srcp2, dstp2, z2)
    h2 = _dense(h1, agg2, degT, W2t, b2r)
    return h2


# speedup vs baseline: 1.0053x; 1.0053x over previous
import functools

import jax
import jax.numpy as jnp
from jax import lax
from jax.experimental import pallas as pl
from jax.experimental.pallas import tpu as pltpu
from jax.experimental.pallas import tpu_sc as plsc

N = 10000
D = 128
N_EDGES = 320000

NUM_TILES = 32
CHUNK = 32
PHASES = 10
CHUNKS_PER_PHASE = 32
CHUNKS_PER_TILE = PHASES * CHUNKS_PER_PHASE
E_PER_TILE = CHUNK * CHUNKS_PER_TILE
EPAD = E_PER_TILE * NUM_TILES
NPAD = 10240
ROWS_PER_TILE = NPAD // 16
NBUF = 8

_MESH = plsc.VectorSubcoreMesh(core_axis_name="c", subcore_axis_name="s")


def _sc_agg_body(table, srcp, dstp, z2, out_agg, src_all, dst_all, *rest):
    rows = rest[:NBUF]
    acc_sh, gsem, ssem = rest[NBUF:]

    c = lax.axis_index("c")
    s = lax.axis_index("s")
    wid = c * 16 + s
    row0 = s * ROWS_PER_TILE

    pltpu.sync_copy(z2.at[pl.ds(row0, ROWS_PER_TILE)],
                    acc_sh.at[pl.ds(row0, ROWS_PER_TILE)])
    plsc.subcore_barrier()

    def drain_scatters():
        for b in range(NBUF):
            pltpu.make_async_copy(z2.at[pl.ds(0, CHUNK)], rows[b],
                                  ssem).wait()

    def group(g, carry):
        j0 = g * NBUF

        @pl.when(g > 0)
        def _():
            drain_scatters()

        handles = [
            pltpu.async_copy(table.at[src_all.at[j0 + b]], rows[b], gsem)
            for b in range(NBUF)
        ]
        for b in range(NBUF):
            handles[b].wait()
            pltpu.async_copy(rows[b], acc_sh.at[dst_all.at[j0 + b]], ssem,
                             add=True)
        return carry

    for phase in range(PHASES):
        c0 = wid * CHUNKS_PER_TILE + phase * CHUNKS_PER_PHASE
        pltpu.sync_copy(srcp.at[pl.ds(c0, CHUNKS_PER_PHASE)], src_all)
        pltpu.sync_copy(dstp.at[pl.ds(c0, CHUNKS_PER_PHASE)], dst_all)
        lax.fori_loop(0, CHUNKS_PER_PHASE // NBUF, group, 0)
        drain_scatters()
    plsc.subcore_barrier()

    pltpu.sync_copy(acc_sh.at[pl.ds(row0, ROWS_PER_TILE)],
                    out_agg.at[c, pl.ds(row0, ROWS_PER_TILE)])


_sc_agg = pl.kernel(
    _sc_agg_body,
    mesh=_MESH,
    out_type=jax.ShapeDtypeStruct((2, NPAD, D), jnp.float32),
    scratch_types=(
        [pltpu.VMEM((CHUNKS_PER_PHASE, CHUNK), jnp.int32),
         pltpu.VMEM((CHUNKS_PER_PHASE, CHUNK), jnp.int32)]
        + [pltpu.VMEM((CHUNK, D), jnp.float32) for _ in range(NBUF)]
        + [pltpu.VMEM_SHARED((NPAD, D), jnp.float32),
           pltpu.SemaphoreType.DMA,
           pltpu.SemaphoreType.DMA]
    ),
)


DCHUNK = 128
DCHUNKS = E_PER_TILE // DCHUNK
DNBUF = 8


def _sc_deg_body(dstp2, z1, out_deg, dst_d, ones_v, deg_sh, sem):
    c = lax.axis_index("c")
    s = lax.axis_index("s")
    wid = c * 16 + s
    row0 = s * ROWS_PER_TILE

    for i in range(DCHUNK // 16):
        ones_v[pl.ds(i * 16, 16)] = jnp.ones((16,), jnp.float32)
    pltpu.sync_copy(z1.at[pl.ds(row0, ROWS_PER_TILE)],
                    deg_sh.at[pl.ds(row0, ROWS_PER_TILE)])
    pltpu.sync_copy(dstp2.at[pl.ds(wid * DCHUNKS, DCHUNKS)], dst_d)
    plsc.subcore_barrier()

    def group(g, carry):
        j0 = g * DNBUF
        handles = [
            pltpu.async_copy(ones_v, deg_sh.at[dst_d.at[j0 + b]], sem,
                             add=True)
            for b in range(DNBUF)
        ]
        for h in handles:
            h.wait()
        return carry

    lax.fori_loop(0, DCHUNKS // DNBUF, group, 0)
    plsc.subcore_barrier()
    pltpu.sync_copy(deg_sh.at[pl.ds(row0, ROWS_PER_TILE)],
                    out_deg.at[c, pl.ds(row0, ROWS_PER_TILE)])


_sc_deg = pl.kernel(
    _sc_deg_body,
    mesh=_MESH,
    out_type=jax.ShapeDtypeStruct((2, NPAD), jnp.float32),
    scratch_types=[
        pltpu.VMEM((DCHUNKS, DCHUNK), jnp.int32),
        pltpu.VMEM((DCHUNK,), jnp.float32),
        pltpu.VMEM_SHARED((NPAD,), jnp.float32),
        pltpu.SemaphoreType.DMA,
    ],
)


def _dense_body(h_ref, agg_ref, degT_ref, wt_ref, b_ref, out_ref):
    h = h_ref[...]
    agg = agg_ref[0] + agg_ref[1]
    d = degT_ref[:, 0:1] + degT_ref[:, 1:2]
    mean = agg * (1.0 / jnp.maximum(d, 1.0))
    z = (jnp.dot(h, wt_ref[0:D], preferred_element_type=jnp.float32)
         + jnp.dot(mean, wt_ref[D:2 * D], preferred_element_type=jnp.float32)
         + b_ref[...])
    a = jnp.where(z >= 0, z, 0.01 * z)
    nrm = jnp.sqrt(jnp.sum(a * a, axis=1, keepdims=True))
    out_ref[...] = a / jnp.maximum(nrm, 1e-6)


def _dense(h, agg, degT, Wt, b2d):
    R = 1024
    return pl.pallas_call(
        _dense_body,
        grid=(NPAD // R,),
        in_specs=[
            pl.BlockSpec((R, D), lambda i: (i, 0)),
            pl.BlockSpec((2, R, D), lambda i: (0, i, 0)),
            pl.BlockSpec((R, 2), lambda i: (i, 0)),
            pl.BlockSpec((2 * D, D), lambda i: (0, 0)),
            pl.BlockSpec((1, D), lambda i: (0, 0)),
        ],
        out_specs=pl.BlockSpec((R, D), lambda i: (i, 0)),
        out_shape=jax.ShapeDtypeStruct((N, D), jnp.float32),
    )(h, agg, degT, Wt, b2d)


def kernel(node_ids, edge_index, node_emb, W1, b1, W2, b2):
    h0 = lax.slice(node_emb, (1, 0), (N + 1, D))

    src = edge_index[0]
    dst = edge_index[1]
    npad_e = EPAD - N_EDGES
    pad_iota = jnp.arange(npad_e, dtype=jnp.int32)
    srcp = jnp.concatenate([src, pad_iota % N])
    dstp = jnp.concatenate([dst, N + pad_iota % (NPAD - N)])
    srcp2 = srcp.reshape(EPAD // CHUNK, CHUNK)
    srcp2e = srcp2 + 1
    dstp2 = dstp.reshape(EPAD // CHUNK, CHUNK)
    dstp2d = dstp.reshape(EPAD // DCHUNK, DCHUNK)

    z2 = jnp.zeros((NPAD, D), jnp.float32)
    z1 = jnp.zeros((NPAD,), jnp.float32)

    W1t = W1.T
    W2t = W2.T
    b1r = b1.reshape(1, D)
    b2r = b2.reshape(1, D)

    deg = _sc_deg(dstp2d, z1)
    degT = deg.T
    agg1 = _sc_agg(node_emb, srcp2e, dstp2, z2)
    h1 = _dense(h0, agg1, degT, W1t, b1r)

    agg2 = _sc_agg(h1, srcp2, dstp2, z2)
    h2 = _dense(h1, agg2, degT, W2t, b2r)
    return h2


# speedup vs baseline: 1.0521x; 1.0465x over previous
import functools

import jax
import jax.numpy as jnp
from jax import lax
from jax.experimental import pallas as pl
from jax.experimental.pallas import tpu as pltpu
from jax.experimental.pallas import tpu_sc as plsc

N = 10000
D = 128
N_EDGES = 320000

NUM_TILES = 32
CHUNK = 64
PHASES = 5
CHUNKS_PER_PHASE = 32
CHUNKS_PER_TILE = PHASES * CHUNKS_PER_PHASE
E_PER_TILE = CHUNK * CHUNKS_PER_TILE
EPAD = E_PER_TILE * NUM_TILES
NPAD = 10240
ROWS_PER_TILE = NPAD // 16
NBUF = 4

_MESH = plsc.VectorSubcoreMesh(core_axis_name="c", subcore_axis_name="s")


def _sc_agg_body(table, srcp, dstp, z2, out_agg, src_all, dst_all, *rest):
    rows = rest[:NBUF]
    acc_sh, gsem, ssem = rest[NBUF:]

    c = lax.axis_index("c")
    s = lax.axis_index("s")
    wid = c * 16 + s
    row0 = s * ROWS_PER_TILE

    pltpu.sync_copy(z2.at[pl.ds(row0, ROWS_PER_TILE)],
                    acc_sh.at[pl.ds(row0, ROWS_PER_TILE)])
    plsc.subcore_barrier()

    def drain_scatters():
        for b in range(NBUF):
            pltpu.make_async_copy(z2.at[pl.ds(0, CHUNK)], rows[b],
                                  ssem).wait()

    def group(g, carry):
        j0 = g * NBUF

        @pl.when(g > 0)
        def _():
            drain_scatters()

        handles = [
            pltpu.async_copy(table.at[src_all.at[j0 + b]], rows[b], gsem)
            for b in range(NBUF)
        ]
        for b in range(NBUF):
            handles[b].wait()
            pltpu.async_copy(rows[b], acc_sh.at[dst_all.at[j0 + b]], ssem,
                             add=True)
        return carry

    for phase in range(PHASES):
        c0 = wid * CHUNKS_PER_TILE + phase * CHUNKS_PER_PHASE
        pltpu.sync_copy(srcp.at[pl.ds(c0, CHUNKS_PER_PHASE)], src_all)
        pltpu.sync_copy(dstp.at[pl.ds(c0, CHUNKS_PER_PHASE)], dst_all)
        lax.fori_loop(0, CHUNKS_PER_PHASE // NBUF, group, 0)
        drain_scatters()
    plsc.subcore_barrier()

    pltpu.sync_copy(acc_sh.at[pl.ds(row0, ROWS_PER_TILE)],
                    out_agg.at[c, pl.ds(row0, ROWS_PER_TILE)])


_sc_agg = pl.kernel(
    _sc_agg_body,
    mesh=_MESH,
    out_type=jax.ShapeDtypeStruct((2, NPAD, D), jnp.float32),
    scratch_types=(
        [pltpu.VMEM((CHUNKS_PER_PHASE, CHUNK), jnp.int32),
         pltpu.VMEM((CHUNKS_PER_PHASE, CHUNK), jnp.int32)]
        + [pltpu.VMEM((CHUNK, D), jnp.float32) for _ in range(NBUF)]
        + [pltpu.VMEM_SHARED((NPAD, D), jnp.float32),
           pltpu.SemaphoreType.DMA,
           pltpu.SemaphoreType.DMA]
    ),
)


DCHUNK = 128
DCHUNKS = E_PER_TILE // DCHUNK
DNBUF = 8


def _sc_deg_body(dstp2, z1, out_deg, dst_d, ones_v, deg_sh, sem):
    c = lax.axis_index("c")
    s = lax.axis_index("s")
    wid = c * 16 + s
    row0 = s * ROWS_PER_TILE

    for i in range(DCHUNK // 16):
        ones_v[pl.ds(i * 16, 16)] = jnp.ones((16,), jnp.float32)
    pltpu.sync_copy(z1.at[pl.ds(row0, ROWS_PER_TILE)],
                    deg_sh.at[pl.ds(row0, ROWS_PER_TILE)])
    pltpu.sync_copy(dstp2.at[pl.ds(wid * DCHUNKS, DCHUNKS)], dst_d)
    plsc.subcore_barrier()

    def group(g, carry):
        j0 = g * DNBUF
        handles = [
            pltpu.async_copy(ones_v, deg_sh.at[dst_d.at[j0 + b]], sem,
                             add=True)
            for b in range(DNBUF)
        ]
        for h in handles:
            h.wait()
        return carry

    lax.fori_loop(0, DCHUNKS // DNBUF, group, 0)
    plsc.subcore_barrier()
    pltpu.sync_copy(deg_sh.at[pl.ds(row0, ROWS_PER_TILE)],
                    out_deg.at[c, pl.ds(row0, ROWS_PER_TILE)])


_sc_deg = pl.kernel(
    _sc_deg_body,
    mesh=_MESH,
    out_type=jax.ShapeDtypeStruct((2, NPAD), jnp.float32),
    scratch_types=[
        pltpu.VMEM((DCHUNKS, DCHUNK), jnp.int32),
        pltpu.VMEM((DCHUNK,), jnp.float32),
        pltpu.VMEM_SHARED((NPAD,), jnp.float32),
        pltpu.SemaphoreType.DMA,
    ],
)


def _dense_body(h_ref, agg_ref, degT_ref, wt_ref, b_ref, out_ref):
    h = h_ref[...]
    agg = agg_ref[0] + agg_ref[1]
    d = degT_ref[:, 0:1] + degT_ref[:, 1:2]
    mean = agg * (1.0 / jnp.maximum(d, 1.0))
    z = (jnp.dot(h, wt_ref[0:D], preferred_element_type=jnp.float32)
         + jnp.dot(mean, wt_ref[D:2 * D], preferred_element_type=jnp.float32)
         + b_ref[...])
    a = jnp.where(z >= 0, z, 0.01 * z)
    nrm = jnp.sqrt(jnp.sum(a * a, axis=1, keepdims=True))
    out_ref[...] = a / jnp.maximum(nrm, 1e-6)


def _dense(h, agg, degT, Wt, b2d):
    R = 1024
    return pl.pallas_call(
        _dense_body,
        grid=(NPAD // R,),
        in_specs=[
            pl.BlockSpec((R, D), lambda i: (i, 0)),
            pl.BlockSpec((2, R, D), lambda i: (0, i, 0)),
            pl.BlockSpec((R, 2), lambda i: (i, 0)),
            pl.BlockSpec((2 * D, D), lambda i: (0, 0)),
            pl.BlockSpec((1, D), lambda i: (0, 0)),
        ],
        out_specs=pl.BlockSpec((R, D), lambda i: (i, 0)),
        out_shape=jax.ShapeDtypeStruct((N, D), jnp.float32),
    )(h, agg, degT, Wt, b2d)


def kernel(node_ids, edge_index, node_emb, W1, b1, W2, b2):
    h0 = lax.slice(node_emb, (1, 0), (N + 1, D))

    src = edge_index[0]
    dst = edge_index[1]
    npad_e = EPAD - N_EDGES
    pad_iota = jnp.arange(npad_e, dtype=jnp.int32)
    srcp = jnp.concatenate([src, pad_iota % N])
    dstp = jnp.concatenate([dst, N + pad_iota % (NPAD - N)])
    srcp2 = srcp.reshape(EPAD // CHUNK, CHUNK)
    srcp2e = srcp2 + 1
    dstp2 = dstp.reshape(EPAD // CHUNK, CHUNK)
    dstp2d = dstp.reshape(EPAD // DCHUNK, DCHUNK)

    z2 = jnp.zeros((NPAD, D), jnp.float32)
    z1 = jnp.zeros((NPAD,), jnp.float32)

    W1t = W1.T
    W2t = W2.T
    b1r = b1.reshape(1, D)
    b2r = b2.reshape(1, D)

    deg = _sc_deg(dstp2d, z1)
    degT = deg.T
    agg1 = _sc_agg(node_emb, srcp2e, dstp2, z2)
    h1 = _dense(h0, agg1, degT, W1t, b1r)

    agg2 = _sc_agg(h1, srcp2, dstp2, z2)
    h2 = _dense(h1, agg2, degT, W2t, b2r)
    return h2
